# Initial kernel scaffold; baseline (speedup 1.0000x reference)
import jax
import jax.numpy as jnp
from jax.experimental import pallas as pl


def kernel(centers, contexts_negatives, embed_v, embed_u):
    raise NotImplementedError("write your pallas kernel here")


# speedup vs baseline: 1.0926x; 1.0926x over previous
import functools

import jax
import jax.numpy as jnp
from jax import lax
from jax.experimental import pallas as pl
from jax.experimental.pallas import tpu as pltpu
from jax.experimental.pallas import tpu_sc as plsc

VOCAB = 1000000
EMBED = 64
B = 16384
L = 50
LANES = 16

NC = 2
NS = 16
NW = NC * NS

C = 16
BPW = B // NW
NCHUNK = BPW // C
GPB = 4
UPAD = 16


def _sc_kernel(ctr_hbm, ctx_hbm, ev_hbm, eu_hbm, out_hbm,
               ctr_idx, ctx_idx, v_rows, u_rows, out_v, sem):
    wid = lax.axis_index("s") * NC + lax.axis_index("c")
    base = wid * BPW

    iota = lax.iota(jnp.int32, LANES)

    def chunk_body(g, _):
        cbase = base + g * C
        pltpu.sync_copy(ctr_hbm.at[pl.ds(cbase, C)], ctr_idx)
        pltpu.sync_copy(ctx_hbm.at[pl.ds(cbase * L, C * L)], ctx_idx)

        copies = [pltpu.async_copy(ev_hbm.at[ctr_idx], v_rows, sem)]
        n_full = (C * L) // 128
        for j in range(n_full):
            copies.append(pltpu.async_copy(
                eu_hbm.at[ctx_idx.at[pl.ds(j * 128, 128)]],
                u_rows.at[pl.ds(j * 128, 128)], sem))
        rem = C * L - n_full * 128
        if rem:
            copies.append(pltpu.async_copy(
                eu_hbm.at[ctx_idx.at[pl.ds(n_full * 128, rem)]],
                u_rows.at[pl.ds(n_full * 128, rem)], sem))
        for cp in copies:
            cp.wait()

        def b_body(b, _):
            vvecs = [v_rows[b, pl.ds(k * LANES, LANES)] for k in range(4)]
            row0 = b * L
            accs = []
            for grp in range(GPB):
                acc = jnp.zeros((LANES,), jnp.float32)
                rows = row0 + grp * LANES + iota
                for e in range(EMBED):
                    vsplat = jnp.take(vvecs[e // LANES],
                                      jnp.full((LANES,), e % LANES, jnp.int32))
                    uvals = plsc.load_gather(
                        u_rows, [rows, jnp.full((LANES,), e, jnp.int32)])
                    acc = acc + uvals * vsplat
                accs.append(acc)
            for grp in range(GPB):
                lane_l = grp * LANES + iota
                plsc.store_scatter(out_v, [row0 + lane_l], accs[grp],
                                   mask=lane_l < L)
            return ()

        lax.fori_loop(0, C, b_body, (), unroll=False)
        pltpu.sync_copy(out_v, out_hbm.at[pl.ds(cbase * L, C * L)])
        return ()

    lax.fori_loop(0, NCHUNK, chunk_body, (), unroll=False)


@jax.jit
def _run(centers_flat, ctx_flat, embed_v, embed_u):
    kfn = pl.kernel(
        _sc_kernel,
        out_type=jax.ShapeDtypeStruct((B * L,), jnp.float32),
        mesh=plsc.VectorSubcoreMesh(core_axis_name="c", subcore_axis_name="s"),
        scratch_types=[
            pltpu.VMEM((C,), jnp.int32),
            pltpu.VMEM((C * L,), jnp.int32),
            pltpu.VMEM((C, EMBED), jnp.float32),
            pltpu.VMEM((C * L + UPAD, EMBED), jnp.float32),
            pltpu.VMEM((C * L,), jnp.float32),
            pltpu.SemaphoreType.DMA,
        ],
        compiler_params=pltpu.CompilerParams(use_tc_tiling_on_sc=False,
                                             needs_layout_passes=False),
    )
    return kfn(centers_flat, ctx_flat, embed_v, embed_u)


def kernel(centers, contexts_negatives, embed_v, embed_u):
    out = _run(centers.reshape(B), contexts_negatives.reshape(B * L),
               embed_v, embed_u)
    return out.reshape(B, 1, L)
